# Initial kernel scaffold; baseline (speedup 1.0000x reference)
import jax
import jax.numpy as jnp
from jax.experimental import pallas as pl


def kernel(x, batch, size, W):
    raise NotImplementedError("write your pallas kernel here")


# speedup vs baseline: 1.9644x; 1.9644x over previous
import functools

import jax
import jax.numpy as jnp
from jax import lax
from jax.experimental import pallas as pl
from jax.experimental.pallas import tpu as pltpu
from jax.experimental.pallas import tpu_sc as plsc

N = 50000
D = 256
S = 512
DG = D // 16

NC = 2
NS = 16
NW = NC * NS

CH = 80
N_CHUNKS = N // CH
K_CH = -(-N_CHUNKS // NW)
S_PAD = 640
ZR = S_PAD // NS

def _zero_dense(xbuf, dst_hbm, c, s):
    zero16 = jnp.zeros((16,), jnp.float32)

    def _zr(r, _):
        for g in range(DG):
            xbuf[r, pl.ds(g * 16, 16)] = zero16
        return 0

    lax.fori_loop(0, ZR, _zr, 0)
    base = c * S_PAD + s * ZR
    pltpu.sync_copy(xbuf.at[pl.ds(0, ZR)], dst_hbm.at[pl.ds(base, ZR)])


def _pass1_body(x_hbm, b_hbm, sums_hbm, cnts_hbm, spill_hbm, spcnt_hbm,
                spid_hbm, xbuf, bbuf2d, cbuf, stage, cstage, istage):
    c = lax.axis_index("c")
    s = lax.axis_index("s")
    w = s * NC + c

    _zero_dense(xbuf, sums_hbm, c, s)
    zero16 = jnp.zeros((16,), jnp.float32)

    def _zc(r, _):
        cbuf[r, :] = zero16
        return 0

    lax.fori_loop(0, ZR, _zc, 0)
    base = c * S_PAD + s * ZR
    pltpu.sync_copy(cbuf, cnts_hbm.at[pl.ds(base, ZR)])
    plsc.subcore_barrier()

    def _flush_dma(cur, first):
        def to_spill(_):
            pltpu.sync_copy(stage, spill_hbm.at[pl.ds(w, 1)])
            pltpu.sync_copy(cstage, spcnt_hbm.at[pl.ds(w, 1)])
            pltpu.sync_copy(istage, spid_hbm.at[pl.ds(w, 1)])
            return 0

        def to_dense(_):
            pltpu.sync_copy(stage, sums_hbm.at[pl.ds(c * S_PAD + cur, 1)])
            pltpu.sync_copy(cstage, cnts_hbm.at[pl.ds(c * S_PAD + cur, 1)])
            return 0

        lax.cond(first > 0, to_spill, to_dense, 0)

    def _row(carry, sid, row, active):
        cur, cnt, first, acc = carry
        newrun = jnp.logical_and(sid != cur, active)
        do_flush = jnp.logical_and(newrun, cur >= 0)

        def flush(_):
            _flush_dma(cur, first)
            return 0

        lax.cond(do_flush, flush, lambda _: 0, 0)
        first = jnp.where(do_flush, jnp.int32(0), first)
        act_f = jnp.where(active, 1.0, 0.0)
        xs = [xbuf[row, pl.ds(g * 16, 16)] for g in range(DG)]
        acc = tuple(
            jnp.where(newrun, 0.0, a) + xv * act_f for a, xv in zip(acc, xs)
        )
        cnt = jnp.where(newrun, 0.0, cnt) + act_f
        cur = jnp.where(active, sid, cur)
        for g in range(DG):
            stage[0, pl.ds(g * 16, 16)] = acc[g]
        cstage[0, :] = jnp.broadcast_to(cnt, (16,))
        istage[0, :] = jnp.where(
            active, jnp.broadcast_to(sid, (16,)), istage[0, :]
        )
        return (cur, cnt, first, acc)

    first_chunk = (N_CHUNKS * w) // NW
    end_chunk = (N_CHUNKS * (w + 1)) // NW

    def _chunk(j, carry):
        cidx = first_chunk + j
        active = cidx < end_chunk
        b = jnp.minimum(cidx, N_CHUNKS - 1) * CH
        pltpu.sync_copy(x_hbm.at[pl.ds(b, CH)], xbuf)
        for G in range(CH // 16):
            pltpu.sync_copy(b_hbm.at[pl.ds(b + G * 16, 16)], bbuf2d.at[G])

        def _group(g, carry):
            bvec = bbuf2d[g, :]
            for l in range(16):
                carry = _row(carry, bvec[l], g * 16 + l, active)
            return carry

        return lax.fori_loop(0, CH // 16, _group, carry)

    init = (
        jnp.int32(-1),
        0.0,
        jnp.int32(1),
        tuple(jnp.zeros((16,), jnp.float32) for _ in range(DG)),
    )
    cur, cnt, first, acc = lax.fori_loop(0, K_CH, _chunk, init)
    _flush_dma(cur, first)


def _mid_body(sums_ref, cnts_ref, spill_ref, spcnt_ref, spid_ref, w_ref, t_ref):
    sums = sums_ref[0:S, :] + sums_ref[S_PAD : S_PAD + S, :]
    cnts = cnts_ref[0:S, 0:1] + cnts_ref[S_PAD : S_PAD + S, 0:1]
    oh = (
        lax.broadcasted_iota(jnp.int32, (S, NW), 0) == spid_ref[:, 0]
    ).astype(jnp.float32)
    sums = sums + jnp.dot(oh, spill_ref[...], preferred_element_type=jnp.float32)
    cnts = cnts + jnp.dot(
        oh, spcnt_ref[...], preferred_element_type=jnp.float32
    )[:, 0:1]
    mean = sums / jnp.maximum(cnts, 1.0)
    t_ref[...] = jnp.tanh(
        jnp.dot(mean, w_ref[...], preferred_element_type=jnp.float32)
    )


_mid = pl.pallas_call(
    _mid_body,
    out_shape=jax.ShapeDtypeStruct((S, D), jnp.float32),
)


def _pass2_body(x_hbm, b_hbm, t_hbm, out_hbm, spill_hbm, spid_hbm,
                xbuf, bbuf2d, tstage, stage, istage):
    c = lax.axis_index("c")
    s = lax.axis_index("s")
    w = s * NC + c

    _zero_dense(xbuf, out_hbm, c, s)
    plsc.subcore_barrier()

    idx16 = lax.iota(jnp.int32, 16)

    def _flush_dma(cur, first):
        def to_spill(_):
            pltpu.sync_copy(stage, spill_hbm.at[pl.ds(w, 1)])
            pltpu.sync_copy(istage, spid_hbm.at[pl.ds(w, 1)])
            return 0

        def to_dense(_):
            pltpu.sync_copy(stage, out_hbm.at[pl.ds(c * S_PAD + cur, 1)])
            return 0

        lax.cond(first > 0, to_spill, to_dense, 0)

    def _row(carry, sid, row, active):
        cur, first, acc = carry
        newrun = jnp.logical_and(sid != cur, active)
        do_flush = jnp.logical_and(newrun, cur >= 0)

        def flush(_):
            _flush_dma(cur, first)
            return 0

        lax.cond(do_flush, flush, lambda _: 0, 0)
        first = jnp.where(do_flush, jnp.int32(0), first)

        @pl.when(newrun)
        def _():
            pltpu.sync_copy(t_hbm.at[pl.ds(sid, 1)], tstage)

        act_f = jnp.where(active, 1.0, 0.0)
        xs = [xbuf[row, pl.ds(g * 16, 16)] for g in range(DG)]
        ts = [tstage[0, pl.ds(g * 16, 16)] for g in range(DG)]
        dot = xs[0] * ts[0]
        for g in range(1, DG):
            dot = dot + xs[g] * ts[g]
        for sh in (8, 4, 2, 1):
            dot = dot + dot[jnp.bitwise_xor(idx16, sh)]
        sig = (1.0 / (1.0 + jnp.exp(-dot))) * act_f
        acc = tuple(
            jnp.where(newrun, 0.0, a) + xv * sig for a, xv in zip(acc, xs)
        )
        cur = jnp.where(active, sid, cur)
        for g in range(DG):
            stage[0, pl.ds(g * 16, 16)] = acc[g]
        istage[0, :] = jnp.where(
            active, jnp.broadcast_to(sid, (16,)), istage[0, :]
        )
        return (cur, first, acc)

    first_chunk = (N_CHUNKS * w) // NW
    end_chunk = (N_CHUNKS * (w + 1)) // NW

    def _chunk(j, carry):
        cidx = first_chunk + j
        active = cidx < end_chunk
        b = jnp.minimum(cidx, N_CHUNKS - 1) * CH
        pltpu.sync_copy(x_hbm.at[pl.ds(b, CH)], xbuf)
        for G in range(CH // 16):
            pltpu.sync_copy(b_hbm.at[pl.ds(b + G * 16, 16)], bbuf2d.at[G])

        def _group(g, carry):
            bvec = bbuf2d[g, :]
            for l in range(16):
                carry = _row(carry, bvec[l], g * 16 + l, active)
            return carry

        return lax.fori_loop(0, CH // 16, _group, carry)

    init = (
        jnp.int32(-1),
        jnp.int32(1),
        tuple(jnp.zeros((16,), jnp.float32) for _ in range(DG)),
    )
    cur, first, acc = lax.fori_loop(0, K_CH, _chunk, init)
    _flush_dma(cur, first)


def _fin_body(q_ref, spill_ref, spid_ref, o_ref):
    dense = q_ref[0:S, :] + q_ref[S_PAD : S_PAD + S, :]
    oh = (
        lax.broadcasted_iota(jnp.int32, (S, NW), 0) == spid_ref[:, 0]
    ).astype(jnp.float32)
    o_ref[...] = dense + jnp.dot(
        oh, spill_ref[...], preferred_element_type=jnp.float32
    )


_fin = pl.pallas_call(
    _fin_body,
    out_shape=jax.ShapeDtypeStruct((S, D), jnp.float32),
)


@functools.lru_cache(maxsize=1)
def _build_sc_kernels():
    mesh = plsc.VectorSubcoreMesh(
        core_axis_name="c", subcore_axis_name="s", num_cores=NC, num_subcores=NS
    )
    p1 = pl.kernel(
        _pass1_body,
        out_type=[
            jax.ShapeDtypeStruct((NC * S_PAD, D), jnp.float32),
            jax.ShapeDtypeStruct((NC * S_PAD, 16), jnp.float32),
            jax.ShapeDtypeStruct((NW, D), jnp.float32),
            jax.ShapeDtypeStruct((NW, 16), jnp.float32),
            jax.ShapeDtypeStruct((NW, 16), jnp.int32),
        ],
        mesh=mesh,
        scratch_types=[
            pltpu.VMEM((CH, D), jnp.float32),
            pltpu.VMEM((CH // 16, 16), jnp.int32),
            pltpu.VMEM((ZR, 16), jnp.float32),
            pltpu.VMEM((1, D), jnp.float32),
            pltpu.VMEM((1, 16), jnp.float32),
            pltpu.VMEM((1, 16), jnp.int32),
        ],
    )
    p2 = pl.kernel(
        _pass2_body,
        out_type=[
            jax.ShapeDtypeStruct((NC * S_PAD, D), jnp.float32),
            jax.ShapeDtypeStruct((NW, D), jnp.float32),
            jax.ShapeDtypeStruct((NW, 16), jnp.int32),
        ],
        mesh=mesh,
        scratch_types=[
            pltpu.VMEM((CH, D), jnp.float32),
            pltpu.VMEM((CH // 16, 16), jnp.int32),
            pltpu.VMEM((1, D), jnp.float32),
            pltpu.VMEM((1, D), jnp.float32),
            pltpu.VMEM((1, 16), jnp.int32),
        ],
    )
    return p1, p2


def kernel(x, batch, size, W):
    p1, p2 = _build_sc_kernels()
    sums, cnts, spill, spcnt, spid = p1(x, batch)
    t = _mid(sums, cnts, spill, spcnt, spid, W)
    parts, spill2, spid2 = p2(x, batch, t)
    return _fin(parts, spill2, spid2)


# speedup vs baseline: 3.2499x; 1.6544x over previous
import functools

import jax
import jax.numpy as jnp
from jax import lax
from jax.experimental import pallas as pl
from jax.experimental.pallas import tpu as pltpu
from jax.experimental.pallas import tpu_sc as plsc

N = 50000
D = 256
S = 512
DG = D // 16

NC = 2
NS = 16
NW = NC * NS

CH = 80
N_CHUNKS = N // CH
K_CH = -(-N_CHUNKS // NW)
S_PAD = 640
ZR = S_PAD // NS

def _zero_dense(xbuf, dst_hbm, c, s):
    zero16 = jnp.zeros((16,), jnp.float32)

    def _zr(r, _):
        for g in range(DG):
            xbuf[r, pl.ds(g * 16, 16)] = zero16
        return 0

    lax.fori_loop(0, ZR, _zr, 0)
    base = c * S_PAD + s * ZR
    pltpu.sync_copy(xbuf.at[pl.ds(0, ZR)], dst_hbm.at[pl.ds(base, ZR)])


def _pass1_body(x_hbm, b_hbm, sums_hbm, cnts_hbm, spill_hbm, spcnt_hbm,
                spid_hbm, xbuf, bbuf2d, cbuf, stage, cstage, istage):
    c = lax.axis_index("c")
    s = lax.axis_index("s")
    w = s * NC + c

    _zero_dense(xbuf, sums_hbm, c, s)
    zero16 = jnp.zeros((16,), jnp.float32)
    idx16 = lax.iota(jnp.int32, 16)

    def _zc(r, _):
        cbuf[r, :] = zero16
        return 0

    lax.fori_loop(0, ZR, _zc, 0)
    base = c * S_PAD + s * ZR
    pltpu.sync_copy(cbuf, cnts_hbm.at[pl.ds(base, ZR)])
    plsc.subcore_barrier()

    def _flush_dma(cur, first):
        def to_spill(_):
            pltpu.sync_copy(stage, spill_hbm.at[pl.ds(w, 1)])
            pltpu.sync_copy(cstage, spcnt_hbm.at[pl.ds(w, 1)])
            pltpu.sync_copy(istage, spid_hbm.at[pl.ds(w, 1)])
            return 0

        def to_dense(_):
            pltpu.sync_copy(stage, sums_hbm.at[pl.ds(c * S_PAD + cur, 1)])
            pltpu.sync_copy(cstage, cnts_hbm.at[pl.ds(c * S_PAD + cur, 1)])
            return 0

        lax.cond(first > 0, to_spill, to_dense, 0)

    def _row(carry, sid, row, active):
        cur, cnt, first, acc = carry
        newrun = jnp.logical_and(sid != cur, active)
        do_flush = jnp.logical_and(newrun, cur >= 0)

        def flush(_):
            _flush_dma(cur, first)
            return 0

        lax.cond(do_flush, flush, lambda _: 0, 0)
        first = jnp.where(do_flush, jnp.int32(0), first)
        act_f = jnp.where(active, 1.0, 0.0)
        xs = [xbuf[row, pl.ds(g * 16, 16)] for g in range(DG)]
        acc = tuple(
            jnp.where(newrun, 0.0, a) + xv * act_f for a, xv in zip(acc, xs)
        )
        cnt = jnp.where(newrun, 0.0, cnt) + act_f
        cur = jnp.where(active, sid, cur)
        for g in range(DG):
            stage[0, pl.ds(g * 16, 16)] = acc[g]
        cstage[0, :] = jnp.broadcast_to(cnt, (16,))
        istage[0, :] = jnp.where(
            active, jnp.broadcast_to(sid, (16,)), istage[0, :]
        )
        return (cur, cnt, first, acc)

    first_chunk = (N_CHUNKS * w) // NW
    end_chunk = (N_CHUNKS * (w + 1)) // NW

    def _chunk(j, carry):
        cidx = first_chunk + j
        active = cidx < end_chunk
        b = jnp.minimum(cidx, N_CHUNKS - 1) * CH
        pltpu.sync_copy(x_hbm.at[pl.ds(b, CH)], xbuf)
        for G in range(CH // 16):
            pltpu.sync_copy(b_hbm.at[pl.ds(b + G * 16, 16)], bbuf2d.at[G])

        def _group(g, carry):
            cur, cnt, first = carry
            bvec = bbuf2d[g, :]
            diff = jnp.bitwise_xor(bvec, jnp.broadcast_to(cur, (16,)))
            for sh in (8, 4, 2, 1):
                diff = jnp.bitwise_or(diff, diff[jnp.bitwise_xor(idx16, sh)])
            fast = jnp.logical_and(diff[0] == 0, active)

            def fast_fn(args):
                cur, cnt, first = args
                accs = [stage[0, pl.ds(gg * 16, 16)] for gg in range(DG)]
                for l in range(16):
                    row = g * 16 + l
                    for gg in range(DG):
                        accs[gg] = accs[gg] + xbuf[row, pl.ds(gg * 16, 16)]
                for gg in range(DG):
                    stage[0, pl.ds(gg * 16, 16)] = accs[gg]
                cnt = cnt + 16.0
                cstage[0, :] = jnp.broadcast_to(cnt, (16,))
                return (cur, cnt, first)

            def slow_fn(args):
                cur, cnt, first = args
                accs = tuple(stage[0, pl.ds(gg * 16, 16)] for gg in range(DG))
                c2 = (cur, cnt, first, accs)
                for l in range(16):
                    c2 = _row(c2, bvec[l], g * 16 + l, active)
                cur, cnt, first, _ = c2
                return (cur, cnt, first)

            return lax.cond(fast, fast_fn, slow_fn, (cur, cnt, first))

        return lax.fori_loop(0, CH // 16, _group, carry)

    zero16i = jnp.zeros((16,), jnp.int32)
    for gg in range(DG):
        stage[0, pl.ds(gg * 16, 16)] = jnp.zeros((16,), jnp.float32)
    cstage[0, :] = jnp.zeros((16,), jnp.float32)
    istage[0, :] = zero16i
    init = (jnp.int32(-1), 0.0, jnp.int32(1))
    cur, cnt, first = lax.fori_loop(0, K_CH, _chunk, init)
    _flush_dma(cur, first)


def _mid_body(sums_ref, cnts_ref, spill_ref, spcnt_ref, spid_ref, w_ref, t_ref):
    sums = sums_ref[0:S, :] + sums_ref[S_PAD : S_PAD + S, :]
    cnts = cnts_ref[0:S, 0:1] + cnts_ref[S_PAD : S_PAD + S, 0:1]
    oh = (
        lax.broadcasted_iota(jnp.int32, (S, NW), 0) == spid_ref[:, 0]
    ).astype(jnp.float32)
    sums = sums + jnp.dot(oh, spill_ref[...], preferred_element_type=jnp.float32)
    cnts = cnts + jnp.dot(
        oh, spcnt_ref[...], preferred_element_type=jnp.float32
    )[:, 0:1]
    mean = sums / jnp.maximum(cnts, 1.0)
    t_ref[...] = jnp.tanh(
        jnp.dot(mean, w_ref[...], preferred_element_type=jnp.float32)
    )


_mid = pl.pallas_call(
    _mid_body,
    out_shape=jax.ShapeDtypeStruct((S, D), jnp.float32),
)


def _pass2_body(x_hbm, b_hbm, t_hbm, out_hbm, spill_hbm, spid_hbm,
                xbuf, bbuf2d, tstage, stage, istage):
    c = lax.axis_index("c")
    s = lax.axis_index("s")
    w = s * NC + c

    _zero_dense(xbuf, out_hbm, c, s)
    plsc.subcore_barrier()

    idx16 = lax.iota(jnp.int32, 16)

    def _flush_dma(cur, first):
        def to_spill(_):
            pltpu.sync_copy(stage, spill_hbm.at[pl.ds(w, 1)])
            pltpu.sync_copy(istage, spid_hbm.at[pl.ds(w, 1)])
            return 0

        def to_dense(_):
            pltpu.sync_copy(stage, out_hbm.at[pl.ds(c * S_PAD + cur, 1)])
            return 0

        lax.cond(first > 0, to_spill, to_dense, 0)

    def _row(carry, sid, row, active):
        cur, first, acc = carry
        newrun = jnp.logical_and(sid != cur, active)
        do_flush = jnp.logical_and(newrun, cur >= 0)

        def flush(_):
            _flush_dma(cur, first)
            return 0

        lax.cond(do_flush, flush, lambda _: 0, 0)
        first = jnp.where(do_flush, jnp.int32(0), first)

        @pl.when(newrun)
        def _():
            pltpu.sync_copy(t_hbm.at[pl.ds(sid, 1)], tstage)

        act_f = jnp.where(active, 1.0, 0.0)
        xs = [xbuf[row, pl.ds(g * 16, 16)] for g in range(DG)]
        ts = [tstage[0, pl.ds(g * 16, 16)] for g in range(DG)]
        sig = _sigdot(xs, ts) * act_f
        acc = tuple(
            jnp.where(newrun, 0.0, a) + xv * sig for a, xv in zip(acc, xs)
        )
        cur = jnp.where(active, sid, cur)
        for g in range(DG):
            stage[0, pl.ds(g * 16, 16)] = acc[g]
        istage[0, :] = jnp.where(
            active, jnp.broadcast_to(sid, (16,)), istage[0, :]
        )
        return (cur, first, acc)

    def _sigdot(xs, ts):
        vs = [x * t for x, t in zip(xs, ts)]
        while len(vs) > 1:
            vs = [a + b for a, b in zip(vs[0::2], vs[1::2])]
        dot = vs[0]
        for sh in (8, 4, 2, 1):
            dot = dot + dot[jnp.bitwise_xor(idx16, sh)]
        return 1.0 / (1.0 + jnp.exp(-dot))

    first_chunk = (N_CHUNKS * w) // NW
    end_chunk = (N_CHUNKS * (w + 1)) // NW

    def _chunk(j, carry):
        cidx = first_chunk + j
        active = cidx < end_chunk
        b = jnp.minimum(cidx, N_CHUNKS - 1) * CH
        pltpu.sync_copy(x_hbm.at[pl.ds(b, CH)], xbuf)
        for G in range(CH // 16):
            pltpu.sync_copy(b_hbm.at[pl.ds(b + G * 16, 16)], bbuf2d.at[G])

        def _group(g, carry):
            cur, first = carry
            bvec = bbuf2d[g, :]
            diff = jnp.bitwise_xor(bvec, jnp.broadcast_to(cur, (16,)))
            for sh in (8, 4, 2, 1):
                diff = jnp.bitwise_or(diff, diff[jnp.bitwise_xor(idx16, sh)])
            fast = jnp.logical_and(diff[0] == 0, active)

            def fast_fn(args):
                cur, first = args
                accs = [stage[0, pl.ds(gg * 16, 16)] for gg in range(DG)]
                ts = [tstage[0, pl.ds(gg * 16, 16)] for gg in range(DG)]
                for l in range(16):
                    row = g * 16 + l
                    xs = [xbuf[row, pl.ds(gg * 16, 16)] for gg in range(DG)]
                    sig = _sigdot(xs, ts)
                    for gg in range(DG):
                        accs[gg] = accs[gg] + xs[gg] * sig
                for gg in range(DG):
                    stage[0, pl.ds(gg * 16, 16)] = accs[gg]
                return (cur, first)

            def slow_fn(args):
                cur, first = args
                accs = tuple(stage[0, pl.ds(gg * 16, 16)] for gg in range(DG))
                c2 = (cur, first, accs)
                for l in range(16):
                    c2 = _row(c2, bvec[l], g * 16 + l, active)
                cur, first, _ = c2
                return (cur, first)

            return lax.cond(fast, fast_fn, slow_fn, (cur, first))

        return lax.fori_loop(0, CH // 16, _group, carry)

    for gg in range(DG):
        stage[0, pl.ds(gg * 16, 16)] = jnp.zeros((16,), jnp.float32)
    istage[0, :] = jnp.zeros((16,), jnp.int32)
    init = (jnp.int32(-1), jnp.int32(1))
    cur, first = lax.fori_loop(0, K_CH, _chunk, init)
    _flush_dma(cur, first)


def _fin_body(q_ref, spill_ref, spid_ref, o_ref):
    dense = q_ref[0:S, :] + q_ref[S_PAD : S_PAD + S, :]
    oh = (
        lax.broadcasted_iota(jnp.int32, (S, NW), 0) == spid_ref[:, 0]
    ).astype(jnp.float32)
    o_ref[...] = dense + jnp.dot(
        oh, spill_ref[...], preferred_element_type=jnp.float32
    )


_fin = pl.pallas_call(
    _fin_body,
    out_shape=jax.ShapeDtypeStruct((S, D), jnp.float32),
)


@functools.lru_cache(maxsize=1)
def _build_sc_kernels():
    mesh = plsc.VectorSubcoreMesh(
        core_axis_name="c", subcore_axis_name="s", num_cores=NC, num_subcores=NS
    )
    p1 = pl.kernel(
        _pass1_body,
        out_type=[
            jax.ShapeDtypeStruct((NC * S_PAD, D), jnp.float32),
            jax.ShapeDtypeStruct((NC * S_PAD, 16), jnp.float32),
            jax.ShapeDtypeStruct((NW, D), jnp.float32),
            jax.ShapeDtypeStruct((NW, 16), jnp.float32),
            jax.ShapeDtypeStruct((NW, 16), jnp.int32),
        ],
        mesh=mesh,
        scratch_types=[
            pltpu.VMEM((CH, D), jnp.float32),
            pltpu.VMEM((CH // 16, 16), jnp.int32),
            pltpu.VMEM((ZR, 16), jnp.float32),
            pltpu.VMEM((1, D), jnp.float32),
            pltpu.VMEM((1, 16), jnp.float32),
            pltpu.VMEM((1, 16), jnp.int32),
        ],
    )
    p2 = pl.kernel(
        _pass2_body,
        out_type=[
            jax.ShapeDtypeStruct((NC * S_PAD, D), jnp.float32),
            jax.ShapeDtypeStruct((NW, D), jnp.float32),
            jax.ShapeDtypeStruct((NW, 16), jnp.int32),
        ],
        mesh=mesh,
        scratch_types=[
            pltpu.VMEM((CH, D), jnp.float32),
            pltpu.VMEM((CH // 16, 16), jnp.int32),
            pltpu.VMEM((1, D), jnp.float32),
            pltpu.VMEM((1, D), jnp.float32),
            pltpu.VMEM((1, 16), jnp.int32),
        ],
    )
    return p1, p2


def kernel(x, batch, size, W):
    p1, p2 = _build_sc_kernels()
    sums, cnts, spill, spcnt, spid = p1(x, batch)
    t = _mid(sums, cnts, spill, spcnt, spid, W)
    parts, spill2, spid2 = p2(x, batch, t)
    return _fin(parts, spill2, spid2)


# speedup vs baseline: 3.3172x; 1.0207x over previous
import functools

import jax
import jax.numpy as jnp
from jax import lax
from jax.experimental import pallas as pl
from jax.experimental.pallas import tpu as pltpu
from jax.experimental.pallas import tpu_sc as plsc

N = 50000
D = 256
S = 512
DG = D // 16

NC = 2
NS = 16
NW = NC * NS

CH = 80
N_CHUNKS = N // CH
K_CH = -(-N_CHUNKS // NW)
S_PAD = 640
ZR = S_PAD // NS

def _zero_dense(xbuf, dst_hbm, c, s):
    zero16 = jnp.zeros((16,), jnp.float32)

    def _zr(r, _):
        for g in range(DG):
            xbuf[r, pl.ds(g * 16, 16)] = zero16
        return 0

    lax.fori_loop(0, ZR, _zr, 0)
    base = c * S_PAD + s * ZR
    pltpu.sync_copy(xbuf.at[pl.ds(0, ZR)], dst_hbm.at[pl.ds(base, ZR)])


def _pass1_body(x_hbm, b_hbm, sums_hbm, cnts_hbm, spill_hbm, spcnt_hbm,
                spid_hbm, xbufA, xbufB, bbufA, bbufB, cbuf, stage, cstage,
                istage, semxA, semxB, semiA, semiB):
    c = lax.axis_index("c")
    s = lax.axis_index("s")
    w = s * NC + c

    _zero_dense(xbufA, sums_hbm, c, s)
    zero16 = jnp.zeros((16,), jnp.float32)
    idx16 = lax.iota(jnp.int32, 16)

    def _zc(r, _):
        cbuf[r, :] = zero16
        return 0

    lax.fori_loop(0, ZR, _zc, 0)
    base = c * S_PAD + s * ZR
    pltpu.sync_copy(cbuf, cnts_hbm.at[pl.ds(base, ZR)])
    plsc.subcore_barrier()

    def _flush_dma(cur, first):
        def to_spill(_):
            pltpu.sync_copy(stage, spill_hbm.at[pl.ds(w, 1)])
            pltpu.sync_copy(cstage, spcnt_hbm.at[pl.ds(w, 1)])
            pltpu.sync_copy(istage, spid_hbm.at[pl.ds(w, 1)])
            return 0

        def to_dense(_):
            pltpu.sync_copy(stage, sums_hbm.at[pl.ds(c * S_PAD + cur, 1)])
            pltpu.sync_copy(cstage, cnts_hbm.at[pl.ds(c * S_PAD + cur, 1)])
            return 0

        lax.cond(first > 0, to_spill, to_dense, 0)

    def _row(carry, sid, row, active, xbuf):
        cur, cnt, first, acc = carry
        newrun = jnp.logical_and(sid != cur, active)
        do_flush = jnp.logical_and(newrun, cur >= 0)

        def flush(_):
            _flush_dma(cur, first)
            return 0

        lax.cond(do_flush, flush, lambda _: 0, 0)
        first = jnp.where(do_flush, jnp.int32(0), first)
        act_f = jnp.where(active, 1.0, 0.0)
        xs = [xbuf[row, pl.ds(g * 16, 16)] for g in range(DG)]
        acc = tuple(
            jnp.where(newrun, 0.0, a) + xv * act_f for a, xv in zip(acc, xs)
        )
        cnt = jnp.where(newrun, 0.0, cnt) + act_f
        cur = jnp.where(active, sid, cur)
        for g in range(DG):
            stage[0, pl.ds(g * 16, 16)] = acc[g]
        cstage[0, :] = jnp.broadcast_to(cnt, (16,))
        istage[0, :] = jnp.where(
            active, jnp.broadcast_to(sid, (16,)), istage[0, :]
        )
        return (cur, cnt, first, acc)

    first_chunk = (N_CHUNKS * w) // NW
    end_chunk = (N_CHUNKS * (w + 1)) // NW

    def _start_load(xb, bb, sx, si, j):
        b = jnp.minimum(first_chunk + j, N_CHUNKS - 1) * CH
        pltpu.async_copy(x_hbm.at[pl.ds(b, CH)], xb, sx)
        for G in range(CH // 16):
            pltpu.async_copy(b_hbm.at[pl.ds(b + G * 16, 16)], bb.at[G], si)

    def _wait_load(xb, bb, sx, si, j):
        b = jnp.minimum(first_chunk + j, N_CHUNKS - 1) * CH
        pltpu.make_async_copy(x_hbm.at[pl.ds(b, CH)], xb, sx).wait()
        for G in range(CH // 16):
            pltpu.make_async_copy(
                b_hbm.at[pl.ds(b + G * 16, 16)], bb.at[G], si
            ).wait()

    def _groups(xbuf, bbuf2d, active, carry):
        def _group(g, carry):
            cur, cnt, first = carry
            bvec = bbuf2d[g, :]
            diff = jnp.bitwise_xor(bvec, jnp.broadcast_to(cur, (16,)))
            for sh in (8, 4, 2, 1):
                diff = jnp.bitwise_or(diff, diff[jnp.bitwise_xor(idx16, sh)])
            fast = jnp.logical_and(diff[0] == 0, active)

            def fast_fn(args):
                cur, cnt, first = args
                accs = [stage[0, pl.ds(gg * 16, 16)] for gg in range(DG)]
                for l in range(16):
                    row = g * 16 + l
                    for gg in range(DG):
                        accs[gg] = accs[gg] + xbuf[row, pl.ds(gg * 16, 16)]
                for gg in range(DG):
                    stage[0, pl.ds(gg * 16, 16)] = accs[gg]
                cnt = cnt + 16.0
                cstage[0, :] = jnp.broadcast_to(cnt, (16,))
                return (cur, cnt, first)

            def slow_fn(args):
                cur, cnt, first = args
                accs = tuple(stage[0, pl.ds(gg * 16, 16)] for gg in range(DG))
                c2 = (cur, cnt, first, accs)
                for l in range(16):
                    c2 = _row(c2, bvec[l], g * 16 + l, active, xbuf)
                cur, cnt, first, _ = c2
                return (cur, cnt, first)

            return lax.cond(fast, fast_fn, slow_fn, (cur, cnt, first))

        return lax.fori_loop(0, CH // 16, _group, carry)

    bufs = ((xbufA, bbufA, semxA, semiA), (xbufB, bbufB, semxB, semiB))
    _start_load(*bufs[0], 0)

    def _cpair(jj, carry):
        for par in (0, 1):
            j2 = jj * 2 + par
            _start_load(*bufs[1 - par], j2 + 1)
            _wait_load(*bufs[par], j2)
            active = (first_chunk + j2) < end_chunk
            carry = _groups(bufs[par][0], bufs[par][1], active, carry)
        return carry

    for gg in range(DG):
        stage[0, pl.ds(gg * 16, 16)] = jnp.zeros((16,), jnp.float32)
    cstage[0, :] = jnp.zeros((16,), jnp.float32)
    istage[0, :] = jnp.zeros((16,), jnp.int32)
    init = (jnp.int32(-1), 0.0, jnp.int32(1))
    cur, cnt, first = lax.fori_loop(0, K_CH // 2, _cpair, init)
    _wait_load(*bufs[0], K_CH)
    _flush_dma(cur, first)


def _mid_body(sums_ref, cnts_ref, spill_ref, spcnt_ref, spid_ref, w_ref, t_ref):
    sums = sums_ref[0:S, :] + sums_ref[S_PAD : S_PAD + S, :]
    cnts = cnts_ref[0:S, 0:1] + cnts_ref[S_PAD : S_PAD + S, 0:1]
    oh = (
        lax.broadcasted_iota(jnp.int32, (S, NW), 0) == spid_ref[:, 0]
    ).astype(jnp.float32)
    sums = sums + jnp.dot(oh, spill_ref[...], preferred_element_type=jnp.float32)
    cnts = cnts + jnp.dot(
        oh, spcnt_ref[...], preferred_element_type=jnp.float32
    )[:, 0:1]
    mean = sums / jnp.maximum(cnts, 1.0)
    t_ref[...] = jnp.tanh(
        jnp.dot(mean, w_ref[...], preferred_element_type=jnp.float32)
    )


_mid = pl.pallas_call(
    _mid_body,
    out_shape=jax.ShapeDtypeStruct((S, D), jnp.float32),
)


def _pass2_body(x_hbm, b_hbm, t_hbm, out_hbm, spill_hbm, spid_hbm,
                xbufA, xbufB, bbufA, bbufB, tstage, stage, istage,
                semxA, semxB, semiA, semiB):
    c = lax.axis_index("c")
    s = lax.axis_index("s")
    w = s * NC + c

    _zero_dense(xbufA, out_hbm, c, s)
    plsc.subcore_barrier()

    idx16 = lax.iota(jnp.int32, 16)

    def _flush_dma(cur, first):
        def to_spill(_):
            pltpu.sync_copy(stage, spill_hbm.at[pl.ds(w, 1)])
            pltpu.sync_copy(istage, spid_hbm.at[pl.ds(w, 1)])
            return 0

        def to_dense(_):
            pltpu.sync_copy(stage, out_hbm.at[pl.ds(c * S_PAD + cur, 1)])
            return 0

        lax.cond(first > 0, to_spill, to_dense, 0)

    def _row(carry, sid, row, active, xbuf):
        cur, first, acc = carry
        newrun = jnp.logical_and(sid != cur, active)
        do_flush = jnp.logical_and(newrun, cur >= 0)

        def flush(_):
            _flush_dma(cur, first)
            return 0

        lax.cond(do_flush, flush, lambda _: 0, 0)
        first = jnp.where(do_flush, jnp.int32(0), first)

        @pl.when(newrun)
        def _():
            pltpu.sync_copy(t_hbm.at[pl.ds(sid, 1)], tstage)

        act_f = jnp.where(active, 1.0, 0.0)
        xs = [xbuf[row, pl.ds(g * 16, 16)] for g in range(DG)]
        ts = [tstage[0, pl.ds(g * 16, 16)] for g in range(DG)]
        sig = _sigdot(xs, ts) * act_f
        acc = tuple(
            jnp.where(newrun, 0.0, a) + xv * sig for a, xv in zip(acc, xs)
        )
        cur = jnp.where(active, sid, cur)
        for g in range(DG):
            stage[0, pl.ds(g * 16, 16)] = acc[g]
        istage[0, :] = jnp.where(
            active, jnp.broadcast_to(sid, (16,)), istage[0, :]
        )
        return (cur, first, acc)

    def _sigdot(xs, ts):
        vs = [x * t for x, t in zip(xs, ts)]
        while len(vs) > 1:
            vs = [a + b for a, b in zip(vs[0::2], vs[1::2])]
        dot = vs[0]
        for sh in (8, 4, 2, 1):
            dot = dot + dot[jnp.bitwise_xor(idx16, sh)]
        return 1.0 / (1.0 + jnp.exp(-dot))

    first_chunk = (N_CHUNKS * w) // NW
    end_chunk = (N_CHUNKS * (w + 1)) // NW

    def _start_load(xb, bb, sx, si, j):
        b = jnp.minimum(first_chunk + j, N_CHUNKS - 1) * CH
        pltpu.async_copy(x_hbm.at[pl.ds(b, CH)], xb, sx)
        for G in range(CH // 16):
            pltpu.async_copy(b_hbm.at[pl.ds(b + G * 16, 16)], bb.at[G], si)

    def _wait_load(xb, bb, sx, si, j):
        b = jnp.minimum(first_chunk + j, N_CHUNKS - 1) * CH
        pltpu.make_async_copy(x_hbm.at[pl.ds(b, CH)], xb, sx).wait()
        for G in range(CH // 16):
            pltpu.make_async_copy(
                b_hbm.at[pl.ds(b + G * 16, 16)], bb.at[G], si
            ).wait()

    def _groups(xbuf, bbuf2d, active, carry):
        def _group(g, carry):
            cur, first = carry
            bvec = bbuf2d[g, :]
            diff = jnp.bitwise_xor(bvec, jnp.broadcast_to(cur, (16,)))
            for sh in (8, 4, 2, 1):
                diff = jnp.bitwise_or(diff, diff[jnp.bitwise_xor(idx16, sh)])
            fast = jnp.logical_and(diff[0] == 0, active)

            def fast_fn(args):
                cur, first = args
                accs = [stage[0, pl.ds(gg * 16, 16)] for gg in range(DG)]
                ts = [tstage[0, pl.ds(gg * 16, 16)] for gg in range(DG)]
                for l in range(16):
                    row = g * 16 + l
                    xs = [xbuf[row, pl.ds(gg * 16, 16)] for gg in range(DG)]
                    sig = _sigdot(xs, ts)
                    for gg in range(DG):
                        accs[gg] = accs[gg] + xs[gg] * sig
                for gg in range(DG):
                    stage[0, pl.ds(gg * 16, 16)] = accs[gg]
                return (cur, first)

            def slow_fn(args):
                cur, first = args
                accs = tuple(stage[0, pl.ds(gg * 16, 16)] for gg in range(DG))
                c2 = (cur, first, accs)
                for l in range(16):
                    c2 = _row(c2, bvec[l], g * 16 + l, active, xbuf)
                cur, first, _ = c2
                return (cur, first)

            return lax.cond(fast, fast_fn, slow_fn, (cur, first))

        return lax.fori_loop(0, CH // 16, _group, carry)

    bufs = ((xbufA, bbufA, semxA, semiA), (xbufB, bbufB, semxB, semiB))
    _start_load(*bufs[0], 0)

    def _cpair(jj, carry):
        for par in (0, 1):
            j2 = jj * 2 + par
            _start_load(*bufs[1 - par], j2 + 1)
            _wait_load(*bufs[par], j2)
            active = (first_chunk + j2) < end_chunk
            carry = _groups(bufs[par][0], bufs[par][1], active, carry)
        return carry

    for gg in range(DG):
        stage[0, pl.ds(gg * 16, 16)] = jnp.zeros((16,), jnp.float32)
    istage[0, :] = jnp.zeros((16,), jnp.int32)
    init = (jnp.int32(-1), jnp.int32(1))
    cur, first = lax.fori_loop(0, K_CH // 2, _cpair, init)
    _wait_load(*bufs[0], K_CH)
    _flush_dma(cur, first)


def _fin_body(q_ref, spill_ref, spid_ref, o_ref):
    dense = q_ref[0:S, :] + q_ref[S_PAD : S_PAD + S, :]
    oh = (
        lax.broadcasted_iota(jnp.int32, (S, NW), 0) == spid_ref[:, 0]
    ).astype(jnp.float32)
    o_ref[...] = dense + jnp.dot(
        oh, spill_ref[...], preferred_element_type=jnp.float32
    )


_fin = pl.pallas_call(
    _fin_body,
    out_shape=jax.ShapeDtypeStruct((S, D), jnp.float32),
)


@functools.lru_cache(maxsize=1)
def _build_sc_kernels():
    mesh = plsc.VectorSubcoreMesh(
        core_axis_name="c", subcore_axis_name="s", num_cores=NC, num_subcores=NS
    )
    p1 = pl.kernel(
        _pass1_body,
        out_type=[
            jax.ShapeDtypeStruct((NC * S_PAD, D), jnp.float32),
            jax.ShapeDtypeStruct((NC * S_PAD, 16), jnp.float32),
            jax.ShapeDtypeStruct((NW, D), jnp.float32),
            jax.ShapeDtypeStruct((NW, 16), jnp.float32),
            jax.ShapeDtypeStruct((NW, 16), jnp.int32),
        ],
        mesh=mesh,
        scratch_types=[
            pltpu.VMEM((CH, D), jnp.float32),
            pltpu.VMEM((CH, D), jnp.float32),
            pltpu.VMEM((CH // 16, 16), jnp.int32),
            pltpu.VMEM((CH // 16, 16), jnp.int32),
            pltpu.VMEM((ZR, 16), jnp.float32),
            pltpu.VMEM((1, D), jnp.float32),
            pltpu.VMEM((1, 16), jnp.float32),
            pltpu.VMEM((1, 16), jnp.int32),
            pltpu.SemaphoreType.DMA,
            pltpu.SemaphoreType.DMA,
            pltpu.SemaphoreType.DMA,
            pltpu.SemaphoreType.DMA,
        ],
    )
    p2 = pl.kernel(
        _pass2_body,
        out_type=[
            jax.ShapeDtypeStruct((NC * S_PAD, D), jnp.float32),
            jax.ShapeDtypeStruct((NW, D), jnp.float32),
            jax.ShapeDtypeStruct((NW, 16), jnp.int32),
        ],
        mesh=mesh,
        scratch_types=[
            pltpu.VMEM((CH, D), jnp.float32),
            pltpu.VMEM((CH, D), jnp.float32),
            pltpu.VMEM((CH // 16, 16), jnp.int32),
            pltpu.VMEM((CH // 16, 16), jnp.int32),
            pltpu.VMEM((1, D), jnp.float32),
            pltpu.VMEM((1, D), jnp.float32),
            pltpu.VMEM((1, 16), jnp.int32),
            pltpu.SemaphoreType.DMA,
            pltpu.SemaphoreType.DMA,
            pltpu.SemaphoreType.DMA,
            pltpu.SemaphoreType.DMA,
        ],
    )
    return p1, p2


def kernel(x, batch, size, W):
    p1, p2 = _build_sc_kernels()
    sums, cnts, spill, spcnt, spid = p1(x, batch)
    t = _mid(sums, cnts, spill, spcnt, spid, W)
    parts, spill2, spid2 = p2(x, batch, t)
    return _fin(parts, spill2, spid2)


# speedup vs baseline: 3.3655x; 1.0146x over previous
import functools

import jax
import jax.numpy as jnp
from jax import lax
from jax.experimental import pallas as pl
from jax.experimental.pallas import tpu as pltpu
from jax.experimental.pallas import tpu_sc as plsc

N = 50000
D = 256
S = 512
DG = D // 16

NC = 2
NS = 16
NW = NC * NS

CH = 80
N_CHUNKS = N // CH
K_CH = -(-N_CHUNKS // NW)
S_PAD = 640
ZR = S_PAD // NS

def _zero_dense(xbuf, dst_hbm, c, s):
    zero16 = jnp.zeros((16,), jnp.float32)

    def _zr(r, _):
        for g in range(DG):
            xbuf[r, pl.ds(g * 16, 16)] = zero16
        return 0

    lax.fori_loop(0, ZR, _zr, 0)
    base = c * S_PAD + s * ZR
    pltpu.sync_copy(xbuf.at[pl.ds(0, ZR)], dst_hbm.at[pl.ds(base, ZR)])


def _pass1_body(x_hbm, b_hbm, sums_hbm, cnts_hbm, spill_hbm, spcnt_hbm,
                spid_hbm, xbufA, xbufB, bbufA, bbufB, cbuf, stage, cstage,
                istage, semxA, semxB, semiA, semiB):
    c = lax.axis_index("c")
    s = lax.axis_index("s")
    w = s * NC + c

    first_chunk = (N_CHUNKS * w) // NW
    end_chunk = (N_CHUNKS * (w + 1)) // NW

    def _start_load(xb, bb, sx, si, j):
        b = jnp.minimum(first_chunk + j, N_CHUNKS - 1) * CH
        pltpu.async_copy(x_hbm.at[pl.ds(b, CH)], xb, sx)
        for G in range(CH // 16):
            pltpu.async_copy(b_hbm.at[pl.ds(b + G * 16, 16)], bb.at[G], si)

    def _wait_load(xb, bb, sx, si, j):
        b = jnp.minimum(first_chunk + j, N_CHUNKS - 1) * CH
        pltpu.make_async_copy(x_hbm.at[pl.ds(b, CH)], xb, sx).wait()
        for G in range(CH // 16):
            pltpu.make_async_copy(
                b_hbm.at[pl.ds(b + G * 16, 16)], bb.at[G], si
            ).wait()

    bufs = ((xbufA, bbufA, semxA, semiA), (xbufB, bbufB, semxB, semiB))
    _start_load(*bufs[0], 0)
    _zero_dense(xbufB, sums_hbm, c, s)
    zero16 = jnp.zeros((16,), jnp.float32)
    idx16 = lax.iota(jnp.int32, 16)

    def _zc(r, _):
        cbuf[r, :] = zero16
        return 0

    lax.fori_loop(0, ZR, _zc, 0)
    base = c * S_PAD + s * ZR
    pltpu.sync_copy(cbuf, cnts_hbm.at[pl.ds(base, ZR)])
    plsc.subcore_barrier()

    def _flush_dma(cur, first):
        def to_spill(_):
            pltpu.sync_copy(stage, spill_hbm.at[pl.ds(w, 1)])
            pltpu.sync_copy(cstage, spcnt_hbm.at[pl.ds(w, 1)])
            pltpu.sync_copy(istage, spid_hbm.at[pl.ds(w, 1)])
            return 0

        def to_dense(_):
            pltpu.sync_copy(stage, sums_hbm.at[pl.ds(c * S_PAD + cur, 1)])
            pltpu.sync_copy(cstage, cnts_hbm.at[pl.ds(c * S_PAD + cur, 1)])
            return 0

        lax.cond(first > 0, to_spill, to_dense, 0)

    def _row(carry, sid, row, active, xbuf):
        cur, cnt, first, acc = carry
        newrun = jnp.logical_and(sid != cur, active)
        do_flush = jnp.logical_and(newrun, cur >= 0)

        def flush(_):
            _flush_dma(cur, first)
            return 0

        lax.cond(do_flush, flush, lambda _: 0, 0)
        first = jnp.where(do_flush, jnp.int32(0), first)
        act_f = jnp.where(active, 1.0, 0.0)
        xs = [xbuf[row, pl.ds(g * 16, 16)] for g in range(DG)]
        acc = tuple(
            jnp.where(newrun, 0.0, a) + xv * act_f for a, xv in zip(acc, xs)
        )
        cnt = jnp.where(newrun, 0.0, cnt) + act_f
        cur = jnp.where(active, sid, cur)
        for g in range(DG):
            stage[0, pl.ds(g * 16, 16)] = acc[g]
        cstage[0, :] = jnp.broadcast_to(cnt, (16,))
        istage[0, :] = jnp.where(
            active, jnp.broadcast_to(sid, (16,)), istage[0, :]
        )
        return (cur, cnt, first, acc)

    def _groups(xbuf, bbuf2d, active, carry):
        def _group(g, carry):
            cur, cnt, first = carry
            bvec = bbuf2d[g, :]
            diff = jnp.bitwise_xor(bvec, jnp.broadcast_to(cur, (16,)))
            for sh in (8, 4, 2, 1):
                diff = jnp.bitwise_or(diff, diff[jnp.bitwise_xor(idx16, sh)])
            fast = jnp.logical_and(diff[0] == 0, active)

            def fast_fn(args):
                cur, cnt, first = args
                accs = [stage[0, pl.ds(gg * 16, 16)] for gg in range(DG)]
                for l in range(16):
                    row = g * 16 + l
                    for gg in range(DG):
                        accs[gg] = accs[gg] + xbuf[row, pl.ds(gg * 16, 16)]
                for gg in range(DG):
                    stage[0, pl.ds(gg * 16, 16)] = accs[gg]
                cnt = cnt + 16.0
                cstage[0, :] = jnp.broadcast_to(cnt, (16,))
                return (cur, cnt, first)

            def slow_fn(args):
                cur, cnt, first = args
                accs = tuple(stage[0, pl.ds(gg * 16, 16)] for gg in range(DG))
                c2 = (cur, cnt, first, accs)
                for l in range(16):
                    c2 = _row(c2, bvec[l], g * 16 + l, active, xbuf)
                cur, cnt, first, _ = c2
                return (cur, cnt, first)

            return lax.cond(fast, fast_fn, slow_fn, (cur, cnt, first))

        return lax.fori_loop(0, CH // 16, _group, carry)

    def _cpair(jj, carry):
        for par in (0, 1):
            j2 = jj * 2 + par
            _start_load(*bufs[1 - par], j2 + 1)
            _wait_load(*bufs[par], j2)
            active = (first_chunk + j2) < end_chunk
            carry = _groups(bufs[par][0], bufs[par][1], active, carry)
        return carry

    for gg in range(DG):
        stage[0, pl.ds(gg * 16, 16)] = jnp.zeros((16,), jnp.float32)
    cstage[0, :] = jnp.zeros((16,), jnp.float32)
    istage[0, :] = jnp.zeros((16,), jnp.int32)
    init = (jnp.int32(-1), 0.0, jnp.int32(1))
    cur, cnt, first = lax.fori_loop(0, K_CH // 2, _cpair, init)
    _wait_load(*bufs[0], K_CH)
    _flush_dma(cur, first)


def _mid_body(sums_ref, cnts_ref, spill_ref, spcnt_ref, spid_ref, w_ref, t_ref):
    sums = sums_ref[0:S, :] + sums_ref[S_PAD : S_PAD + S, :]
    cnts = cnts_ref[0:S, 0:1] + cnts_ref[S_PAD : S_PAD + S, 0:1]
    oh = (
        lax.broadcasted_iota(jnp.int32, (S, NW), 0) == spid_ref[:, 0]
    ).astype(jnp.float32)
    sums = sums + jnp.dot(oh, spill_ref[...], preferred_element_type=jnp.float32)
    cnts = cnts + jnp.dot(
        oh, spcnt_ref[...], preferred_element_type=jnp.float32
    )[:, 0:1]
    mean = sums / jnp.maximum(cnts, 1.0)
    t_ref[...] = jnp.tanh(
        jnp.dot(mean, w_ref[...], preferred_element_type=jnp.float32)
    )


_mid = pl.pallas_call(
    _mid_body,
    out_shape=jax.ShapeDtypeStruct((S, D), jnp.float32),
)


def _pass2_body(x_hbm, b_hbm, t_hbm, out_hbm, spill_hbm, spid_hbm,
                xbufA, xbufB, bbufA, bbufB, tstage, stage, istage,
                semxA, semxB, semiA, semiB):
    c = lax.axis_index("c")
    s = lax.axis_index("s")
    w = s * NC + c

    first_chunk = (N_CHUNKS * w) // NW
    end_chunk = (N_CHUNKS * (w + 1)) // NW

    def _start_load(xb, bb, sx, si, j):
        b = jnp.minimum(first_chunk + j, N_CHUNKS - 1) * CH
        pltpu.async_copy(x_hbm.at[pl.ds(b, CH)], xb, sx)
        for G in range(CH // 16):
            pltpu.async_copy(b_hbm.at[pl.ds(b + G * 16, 16)], bb.at[G], si)

    def _wait_load(xb, bb, sx, si, j):
        b = jnp.minimum(first_chunk + j, N_CHUNKS - 1) * CH
        pltpu.make_async_copy(x_hbm.at[pl.ds(b, CH)], xb, sx).wait()
        for G in range(CH // 16):
            pltpu.make_async_copy(
                b_hbm.at[pl.ds(b + G * 16, 16)], bb.at[G], si
            ).wait()

    bufs = ((xbufA, bbufA, semxA, semiA), (xbufB, bbufB, semxB, semiB))
    _start_load(*bufs[0], 0)
    _zero_dense(xbufB, out_hbm, c, s)
    plsc.subcore_barrier()

    idx16 = lax.iota(jnp.int32, 16)

    def _flush_dma(cur, first):
        def to_spill(_):
            pltpu.sync_copy(stage, spill_hbm.at[pl.ds(w, 1)])
            pltpu.sync_copy(istage, spid_hbm.at[pl.ds(w, 1)])
            return 0

        def to_dense(_):
            pltpu.sync_copy(stage, out_hbm.at[pl.ds(c * S_PAD + cur, 1)])
            return 0

        lax.cond(first > 0, to_spill, to_dense, 0)

    def _row(carry, sid, row, active, xbuf):
        cur, first, acc = carry
        newrun = jnp.logical_and(sid != cur, active)
        do_flush = jnp.logical_and(newrun, cur >= 0)

        def flush(_):
            _flush_dma(cur, first)
            return 0

        lax.cond(do_flush, flush, lambda _: 0, 0)
        first = jnp.where(do_flush, jnp.int32(0), first)

        @pl.when(newrun)
        def _():
            pltpu.sync_copy(t_hbm.at[pl.ds(sid, 1)], tstage)

        act_f = jnp.where(active, 1.0, 0.0)
        xs = [xbuf[row, pl.ds(g * 16, 16)] for g in range(DG)]
        ts = [tstage[0, pl.ds(g * 16, 16)] for g in range(DG)]
        sig = _sigdot(xs, ts) * act_f
        acc = tuple(
            jnp.where(newrun, 0.0, a) + xv * sig for a, xv in zip(acc, xs)
        )
        cur = jnp.where(active, sid, cur)
        for g in range(DG):
            stage[0, pl.ds(g * 16, 16)] = acc[g]
        istage[0, :] = jnp.where(
            active, jnp.broadcast_to(sid, (16,)), istage[0, :]
        )
        return (cur, first, acc)

    def _sigdot(xs, ts):
        vs = [x * t for x, t in zip(xs, ts)]
        while len(vs) > 1:
            vs = [a + b for a, b in zip(vs[0::2], vs[1::2])]
        dot = vs[0]
        for sh in (8, 4, 2, 1):
            dot = dot + dot[jnp.bitwise_xor(idx16, sh)]
        return 1.0 / (1.0 + jnp.exp(-dot))

    def _groups(xbuf, bbuf2d, active, carry):
        def _group(g, carry):
            cur, first = carry
            bvec = bbuf2d[g, :]
            diff = jnp.bitwise_xor(bvec, jnp.broadcast_to(cur, (16,)))
            for sh in (8, 4, 2, 1):
                diff = jnp.bitwise_or(diff, diff[jnp.bitwise_xor(idx16, sh)])
            fast = jnp.logical_and(diff[0] == 0, active)

            def fast_fn(args):
                cur, first = args
                accs = [stage[0, pl.ds(gg * 16, 16)] for gg in range(DG)]
                ts = [tstage[0, pl.ds(gg * 16, 16)] for gg in range(DG)]
                for l in range(16):
                    row = g * 16 + l
                    xs = [xbuf[row, pl.ds(gg * 16, 16)] for gg in range(DG)]
                    sig = _sigdot(xs, ts)
                    for gg in range(DG):
                        accs[gg] = accs[gg] + xs[gg] * sig
                for gg in range(DG):
                    stage[0, pl.ds(gg * 16, 16)] = accs[gg]
                return (cur, first)

            def slow_fn(args):
                cur, first = args
                accs = tuple(stage[0, pl.ds(gg * 16, 16)] for gg in range(DG))
                c2 = (cur, first, accs)
                for l in range(16):
                    c2 = _row(c2, bvec[l], g * 16 + l, active, xbuf)
                cur, first, _ = c2
                return (cur, first)

            return lax.cond(fast, fast_fn, slow_fn, (cur, first))

        return lax.fori_loop(0, CH // 16, _group, carry)

    def _cpair(jj, carry):
        for par in (0, 1):
            j2 = jj * 2 + par
            _start_load(*bufs[1 - par], j2 + 1)
            _wait_load(*bufs[par], j2)
            active = (first_chunk + j2) < end_chunk
            carry = _groups(bufs[par][0], bufs[par][1], active, carry)
        return carry

    for gg in range(DG):
        stage[0, pl.ds(gg * 16, 16)] = jnp.zeros((16,), jnp.float32)
    istage[0, :] = jnp.zeros((16,), jnp.int32)
    init = (jnp.int32(-1), jnp.int32(1))
    cur, first = lax.fori_loop(0, K_CH // 2, _cpair, init)
    _wait_load(*bufs[0], K_CH)
    _flush_dma(cur, first)


def _fin_body(q_ref, spill_ref, spid_ref, o_ref):
    dense = q_ref[0:S, :] + q_ref[S_PAD : S_PAD + S, :]
    oh = (
        lax.broadcasted_iota(jnp.int32, (S, NW), 0) == spid_ref[:, 0]
    ).astype(jnp.float32)
    o_ref[...] = dense + jnp.dot(
        oh, spill_ref[...], preferred_element_type=jnp.float32
    )


_fin = pl.pallas_call(
    _fin_body,
    out_shape=jax.ShapeDtypeStruct((S, D), jnp.float32),
)


@functools.lru_cache(maxsize=1)
def _build_sc_kernels():
    mesh = plsc.VectorSubcoreMesh(
        core_axis_name="c", subcore_axis_name="s", num_cores=NC, num_subcores=NS
    )
    p1 = pl.kernel(
        _pass1_body,
        out_type=[
            jax.ShapeDtypeStruct((NC * S_PAD, D), jnp.float32),
            jax.ShapeDtypeStruct((NC * S_PAD, 16), jnp.float32),
            jax.ShapeDtypeStruct((NW, D), jnp.float32),
            jax.ShapeDtypeStruct((NW, 16), jnp.float32),
            jax.ShapeDtypeStruct((NW, 16), jnp.int32),
        ],
        mesh=mesh,
        scratch_types=[
            pltpu.VMEM((CH, D), jnp.float32),
            pltpu.VMEM((CH, D), jnp.float32),
            pltpu.VMEM((CH // 16, 16), jnp.int32),
            pltpu.VMEM((CH // 16, 16), jnp.int32),
            pltpu.VMEM((ZR, 16), jnp.float32),
            pltpu.VMEM((1, D), jnp.float32),
            pltpu.VMEM((1, 16), jnp.float32),
            pltpu.VMEM((1, 16), jnp.int32),
            pltpu.SemaphoreType.DMA,
            pltpu.SemaphoreType.DMA,
            pltpu.SemaphoreType.DMA,
            pltpu.SemaphoreType.DMA,
        ],
    )
    p2 = pl.kernel(
        _pass2_body,
        out_type=[
            jax.ShapeDtypeStruct((NC * S_PAD, D), jnp.float32),
            jax.ShapeDtypeStruct((NW, D), jnp.float32),
            jax.ShapeDtypeStruct((NW, 16), jnp.int32),
        ],
        mesh=mesh,
        scratch_types=[
            pltpu.VMEM((CH, D), jnp.float32),
            pltpu.VMEM((CH, D), jnp.float32),
            pltpu.VMEM((CH // 16, 16), jnp.int32),
            pltpu.VMEM((CH // 16, 16), jnp.int32),
            pltpu.VMEM((1, D), jnp.float32),
            pltpu.VMEM((1, D), jnp.float32),
            pltpu.VMEM((1, 16), jnp.int32),
            pltpu.SemaphoreType.DMA,
            pltpu.SemaphoreType.DMA,
            pltpu.SemaphoreType.DMA,
            pltpu.SemaphoreType.DMA,
        ],
    )
    return p1, p2


def kernel(x, batch, size, W):
    p1, p2 = _build_sc_kernels()
    sums, cnts, spill, spcnt, spid = p1(x, batch)
    t = _mid(sums, cnts, spill, spcnt, spid, W)
    parts, spill2, spid2 = p2(x, batch, t)
    return _fin(parts, spill2, spid2)
